# Initial kernel scaffold; baseline (speedup 1.0000x reference)
import jax
import jax.numpy as jnp
from jax import lax
from jax.experimental import pallas as pl
from jax.experimental.pallas import tpu as pltpu
from jax.experimental.pallas import tpu_sc as plsc

B = 16
N = 128
FEAT = 9
VOCAB = 128
N_HEAD = 32
N_EMBD = 768
ROWS_PER_B = N * FEAT + FEAT
ATOM_ROWS = N * FEAT

OFF_IN = ATOM_ROWS
OFF_OUT = OFF_IN + 512
ROW_GT = OFF_OUT + 512
ROW_ZERO = ROW_GT + 1

W_ROWS = 608
CHUNK = 32
N_CHUNKS = W_ROWS // CHUNK
P1_BASE = ROWS_PER_B - W_ROWS

REL_V = 144
EDGE_V = 512
VOC = REL_V + EDGE_V
QCHUNK = 16
CCOLS = QCHUNK * N
N_QC = N // QCHUNK


def _sc_node_kernel(wcat_hbm, ia_hbm, i1_hbm, i2_hbm, out_hbm,
                    ia_v, i1_v, i2_v, buf_a, buf_1, buf_2, sem):
    w = lax.axis_index("s") * 2 + lax.axis_index("c")
    b = w // 2
    base = (w % 2) * P1_BASE

    pltpu.sync_copy(ia_hbm.at[w], ia_v)
    pltpu.sync_copy(i1_hbm.at[w], i1_v)
    pltpu.sync_copy(i2_hbm.at[w], i2_v)

    @pl.loop(0, N_CHUNKS)
    def _chunk(c):
        r0 = pl.multiple_of(c * CHUNK, CHUNK)
        h1 = pltpu.async_copy(wcat_hbm.at[ia_v.at[pl.ds(r0, CHUNK)]], buf_a, sem)
        h2 = pltpu.async_copy(wcat_hbm.at[i1_v.at[pl.ds(r0, CHUNK)]], buf_1, sem)
        h3 = pltpu.async_copy(wcat_hbm.at[i2_v.at[pl.ds(r0, CHUNK)]], buf_2, sem)
        h1.wait()
        h2.wait()
        h3.wait()

        @pl.loop(0, CHUNK)
        def _row(r):
            @pl.loop(0, N_EMBD, step=64)
            def _vec(j):
                for u in range(4):
                    sl = pl.ds(j + u * 16, 16)
                    buf_a[r, sl] = buf_a[r, sl] + buf_1[r, sl] + buf_2[r, sl]

        pltpu.sync_copy(buf_a, out_hbm.at[b, pl.ds(base + r0, CHUNK)])


def _sc_node(wcat, ia, i1, i2):
    mesh = plsc.VectorSubcoreMesh(core_axis_name="c", subcore_axis_name="s")
    kern = pl.kernel(
        _sc_node_kernel,
        out_type=jax.ShapeDtypeStruct((B, ROWS_PER_B, N_EMBD), jnp.float32),
        mesh=mesh,
        scratch_types=[
            pltpu.VMEM((W_ROWS,), jnp.int32),
            pltpu.VMEM((W_ROWS,), jnp.int32),
            pltpu.VMEM((W_ROWS,), jnp.int32),
            pltpu.VMEM((CHUNK, N_EMBD), jnp.float32),
            pltpu.VMEM((CHUNK, N_EMBD), jnp.float32),
            pltpu.VMEM((CHUNK, N_EMBD), jnp.float32),
            pltpu.SemaphoreType.DMA,
        ],
    )
    return kern(wcat, ia, i1, i2)


def _gab_body(ra_ref, e0_ref, e1_ref, e2_ref, ab_ref, wc_ref, wvt_ref, out_ref):
    kc = pl.program_id(1)

    ra = ra_ref[0, 0]
    iv_r = lax.broadcasted_iota(jnp.int32, (REL_V, CCOLS), 0)
    one = jnp.bfloat16(1.0)
    zero = jnp.bfloat16(0.0)
    oh_rel = jnp.where(iv_r == ra, one, zero)

    iv_e = lax.broadcasted_iota(jnp.int32, (EDGE_V, CCOLS), 0)
    oh_e = (jnp.where(iv_e == e0_ref[0, 0], one, zero)
            + jnp.where(iv_e == e1_ref[0, 0], one, zero)
            + jnp.where(iv_e == e2_ref[0, 0], one, zero))

    oh = jnp.concatenate([oh_rel, oh_e], axis=0)
    acc = lax.dot_general(wc_ref[...], oh, (((1,), (0,)), ((), ())),
                          preferred_element_type=jnp.float32)
    t = acc.reshape(N_HEAD, QCHUNK, N)

    abr = ab_ref[0, pl.ds(1 + kc * QCHUNK, QCHUNK), :]
    col0 = 2.0 * abr[None, :, 0:1] + wvt_ref[...][:, None, :]
    inter = 2.0 * abr[None, :, 1:] + t
    rows = jnp.concatenate([col0, inter], axis=2)
    out_ref[0, :, pl.ds(1 + kc * QCHUNK, QCHUNK), :] = rows

    @pl.when(kc == 0)
    def _():
        out_ref[0, :, 0, :] = jnp.broadcast_to(2.0 * ab_ref[0, 0:1, :],
                                               (N_HEAD, N + 1))


def _gab_call(ra, e0, e1, e2, ab, wct, wvt, interpret=False):
    idx_spec = pl.BlockSpec((1, 1, 1, CCOLS), lambda b, k: (b, k, 0, 0))
    return pl.pallas_call(
        _gab_body,
        grid=(B, N_QC),
        in_specs=[
            idx_spec, idx_spec, idx_spec, idx_spec,
            pl.BlockSpec((1, N + 1, N + 1), lambda b, k: (b, 0, 0)),
            pl.BlockSpec((N_HEAD, VOC), lambda b, k: (0, 0)),
            pl.BlockSpec((N_HEAD, 1), lambda b, k: (0, 0)),
        ],
        out_specs=pl.BlockSpec((1, N_HEAD, N + 1, N + 1),
                               lambda b, k: (b, 0, 0, 0)),
        out_shape=jax.ShapeDtypeStruct((B, N_HEAD, N + 1, N + 1), jnp.float32),
        interpret=interpret,
    )(ra, e0, e1, e2, ab, wct, wvt)


def _prep_node_inputs(x, in_degree, out_degree, W_atom, W_in, W_out, W_gt):
    x = x.astype(jnp.int32)
    t = jnp.arange(ROWS_PER_B, dtype=jnp.int32)
    node = jnp.minimum(t // FEAT, N - 1)
    feat = t % FEAT
    is_atom = t < ATOM_ROWS

    xo = x + (jnp.arange(FEAT, dtype=jnp.int32) * VOCAB)
    a_rows = xo[:, node, feat]
    ia = jnp.where(is_atom[None, :], a_rows, ROW_GT)
    i1 = jnp.where(is_atom[None, :],
                   OFF_IN + in_degree.astype(jnp.int32)[:, node], ROW_ZERO)
    i2 = jnp.where(is_atom[None, :],
                   OFF_OUT + out_degree.astype(jnp.int32)[:, node], ROW_ZERO)

    starts = jnp.array([0, P1_BASE], dtype=jnp.int32)
    rows = starts[:, None] + jnp.arange(W_ROWS, dtype=jnp.int32)[None, :]

    def per_batch(v):
        return v[:, rows].reshape(B * 2, W_ROWS)

    wcat = jnp.concatenate([
        W_atom[:ATOM_ROWS],
        W_in, W_out, W_gt,
        jnp.zeros((1, N_EMBD), dtype=jnp.float32),
    ], axis=0)
    return wcat, per_batch(ia), per_batch(i1), per_batch(i2)


def _prep_gab_inputs(rel_pos, attn_edge_type, W_rel_pos, W_edge, W_vt):
    ra = rel_pos.astype(jnp.int32).reshape(B, N_QC, 1, CCOLS)
    aet = attn_edge_type.astype(jnp.int32)
    e0 = aet[..., 0].reshape(B, N_QC, 1, CCOLS)
    e1 = aet[..., 1].reshape(B, N_QC, 1, CCOLS)
    e2 = aet[..., 2].reshape(B, N_QC, 1, CCOLS)

    wc = jnp.zeros((VOC, N_HEAD), dtype=jnp.float32)
    wc = wc.at[: VOCAB + 1].set(W_rel_pos)
    wc = wc.at[REL_V:].set(W_edge * (1.0 / 3.0))
    wct = wc.T.astype(jnp.bfloat16)
    wvt = W_vt.reshape(1, N_HEAD).T
    return ra, e0, e1, e2, wct, wvt


def kernel(x, y, attn_bias, rel_pos, in_degree, out_degree, edge_input,
           attn_edge_type, W_rel_pos, W_vt, W_edge, W_atom, W_in, W_out,
           W_gt):
    wcat, ia, i1, i2 = _prep_node_inputs(
        x, in_degree, out_degree, W_atom, W_in, W_out, W_gt)
    gnf = _sc_node(wcat, ia, i1, i2)

    ra, e0, e1, e2, wct, wvt = _prep_gab_inputs(
        rel_pos, attn_edge_type, W_rel_pos, W_edge, W_vt)
    gab = _gab_call(ra, e0, e1, e2, attn_bias, wct, wvt)
    return (gnf, gab)


# speedup vs baseline: 15.6058x; 15.6058x over previous
import jax
import jax.numpy as jnp
from jax import lax
from jax.experimental import pallas as pl
from jax.experimental.pallas import tpu as pltpu
from jax.experimental.pallas import tpu_sc as plsc

B = 16
N = 128
FEAT = 9
VOCAB = 128
N_HEAD = 32
N_EMBD = 768
ROWS_PER_B = N * FEAT + FEAT
ATOM_ROWS = N * FEAT

OFF_IN = ATOM_ROWS
OFF_OUT = OFF_IN + 512
ROW_GT = OFF_OUT + 512
ROW_ZERO = ROW_GT + 1

W_ROWS = 608
CHUNK = 32
N_CHUNKS = W_ROWS // CHUNK
P1_BASE = 552
LAST_ROW = ROWS_PER_B - 1

REL_V = 144
EDGE_V = 512
VOC = REL_V + EDGE_V
QCHUNK = 16
CCOLS = QCHUNK * N
N_QC = N // QCHUNK


def _sc_node_kernel(wcat_hbm, ia_hbm, i1_hbm, i2_hbm, out_hbm,
                    ia_v, i1_v, i2_v, buf_a, buf_1, buf_2, sem):
    w = lax.axis_index("s") * 2 + lax.axis_index("c")
    b = w // 2
    base = (w % 2) * P1_BASE

    pltpu.sync_copy(ia_hbm.at[w], ia_v)
    pltpu.sync_copy(i1_hbm.at[w], i1_v)
    pltpu.sync_copy(i2_hbm.at[w], i2_v)

    @pl.loop(0, N_CHUNKS)
    def _chunk(c):
        r0 = pl.multiple_of(c * CHUNK, CHUNK)
        h1 = pltpu.async_copy(wcat_hbm.at[ia_v.at[pl.ds(r0, CHUNK)]], buf_a, sem)
        h2 = pltpu.async_copy(wcat_hbm.at[i1_v.at[pl.ds(r0, CHUNK)]], buf_1, sem)
        h3 = pltpu.async_copy(wcat_hbm.at[i2_v.at[pl.ds(r0, CHUNK)]], buf_2, sem)
        h1.wait()
        h2.wait()
        h3.wait()

        @pl.loop(0, CHUNK)
        def _row(r):
            @pl.loop(0, N_EMBD, step=64)
            def _vec(j):
                for u in range(4):
                    sl = pl.ds(j + u * 16, 16)
                    buf_a[r, sl] = buf_a[r, sl] + buf_1[r, sl] + buf_2[r, sl]

        pltpu.sync_copy(buf_a, out_hbm.at[b, pl.ds(base + r0, CHUNK)])

    @pl.when(w % 2 == 1)
    def _last_row():
        pltpu.sync_copy(wcat_hbm.at[pl.ds(ROW_GT, 1)],
                        out_hbm.at[b, pl.ds(LAST_ROW, 1)])


def _sc_node(wcat, ia, i1, i2):
    mesh = plsc.VectorSubcoreMesh(core_axis_name="c", subcore_axis_name="s")
    kern = pl.kernel(
        _sc_node_kernel,
        out_type=jax.ShapeDtypeStruct((B, ROWS_PER_B, N_EMBD), jnp.float32),
        mesh=mesh,
        scratch_types=[
            pltpu.VMEM((W_ROWS,), jnp.int32),
            pltpu.VMEM((W_ROWS,), jnp.int32),
            pltpu.VMEM((W_ROWS,), jnp.int32),
            pltpu.VMEM((CHUNK, N_EMBD), jnp.float32),
            pltpu.VMEM((CHUNK, N_EMBD), jnp.float32),
            pltpu.VMEM((CHUNK, N_EMBD), jnp.float32),
            pltpu.SemaphoreType.DMA,
        ],
    )
    return kern(wcat, ia, i1, i2)


def _multi_hot(ra, e0, e1, e2, cols):
    one = jnp.bfloat16(1.0)
    zero = jnp.bfloat16(0.0)
    iv_r = lax.broadcasted_iota(jnp.int16, (REL_V, cols), 0)
    oh_rel = jnp.where(iv_r == ra, one, zero)
    iv_e = lax.broadcasted_iota(jnp.int16, (EDGE_V, cols), 0)
    oh_e = (jnp.where(iv_e == e0, one, zero)
            + jnp.where(iv_e == e1, one, zero)
            + jnp.where(iv_e == e2, one, zero))
    return jnp.concatenate([oh_rel, oh_e], axis=0)


def _gab_body(ra_ref, e0_ref, e1_ref, e2_ref, rt_ref, et0_ref, et1_ref,
              et2_ref, ab_ref, wc_ref, wvt_ref, out_ref):
    kc = pl.program_id(1)

    oh = _multi_hot(ra_ref[0, 0], e0_ref[0, 0], e1_ref[0, 0], e2_ref[0, 0],
                    CCOLS)
    acc = lax.dot_general(wc_ref[...], oh, (((1,), (0,)), ((), ())),
                          preferred_element_type=jnp.float32)
    t = acc.reshape(N_HEAD, QCHUNK, N)

    abq = ab_ref[0, pl.ds(kc * QCHUNK, QCHUNK), :]
    notrow0 = (lax.broadcasted_iota(jnp.int32, (QCHUNK, 1), 0)
               + kc * QCHUNK) > 0
    vtc = jnp.where(notrow0[None, :, :], wvt_ref[...][:, None, :], 0.0)
    col0 = 2.0 * abq[None, :, 0:1] + vtc
    inter = 2.0 * abq[None, :, 1:] + t
    rows = jnp.concatenate([col0, inter], axis=2)
    out_ref[0, :, pl.ds(kc * QCHUNK, QCHUNK), :] = rows

    @pl.when(kc == N_QC - 1)
    def _tail():
        oh_t = _multi_hot(rt_ref[0, 0], et0_ref[0, 0], et1_ref[0, 0],
                          et2_ref[0, 0], N)
        acc_t = lax.dot_general(wc_ref[...], oh_t, (((1,), (0,)), ((), ())),
                                preferred_element_type=jnp.float32)
        ab_l = ab_ref[0, pl.ds(N, 1), :]
        c0 = 2.0 * ab_l[:, 0:1] + wvt_ref[...]
        it = 2.0 * ab_l[:, 1:] + acc_t
        out_ref[0, :, N, :] = jnp.concatenate([c0, it], axis=1)


def _gab_call(ra, e0, e1, e2, rt, et0, et1, et2, ab, wct, wvt,
              interpret=False):
    idx_spec = pl.BlockSpec((1, 1, 1, CCOLS), lambda b, k: (b, k, 0, 0))
    tail_spec = pl.BlockSpec((1, 1, N), lambda b, k: (b, 0, 0))
    return pl.pallas_call(
        _gab_body,
        grid=(B, N_QC),
        in_specs=[
            idx_spec, idx_spec, idx_spec, idx_spec,
            tail_spec, tail_spec, tail_spec, tail_spec,
            pl.BlockSpec((1, N + 1, N + 1), lambda b, k: (b, 0, 0)),
            pl.BlockSpec((N_HEAD, VOC), lambda b, k: (0, 0)),
            pl.BlockSpec((N_HEAD, 1), lambda b, k: (0, 0)),
        ],
        out_specs=pl.BlockSpec((1, N_HEAD, N + 1, N + 1),
                               lambda b, k: (b, 0, 0, 0)),
        out_shape=jax.ShapeDtypeStruct((B, N_HEAD, N + 1, N + 1), jnp.float32),
        interpret=interpret,
    )(ra, e0, e1, e2, rt, et0, et1, et2, ab, wct, wvt)


def _prep_node_inputs(x, in_degree, out_degree, W_atom, W_in, W_out, W_gt):
    x = x.astype(jnp.int32)
    t = jnp.arange(ROWS_PER_B, dtype=jnp.int32)
    node = jnp.minimum(t // FEAT, N - 1)
    feat = t % FEAT
    is_atom = t < ATOM_ROWS

    xo = x + (jnp.arange(FEAT, dtype=jnp.int32) * VOCAB)
    a_rows = xo[:, node, feat]
    ia = jnp.where(is_atom[None, :], a_rows, ROW_GT)
    i1 = jnp.where(is_atom[None, :],
                   OFF_IN + in_degree.astype(jnp.int32)[:, node], ROW_ZERO)
    i2 = jnp.where(is_atom[None, :],
                   OFF_OUT + out_degree.astype(jnp.int32)[:, node], ROW_ZERO)

    starts = jnp.array([0, P1_BASE], dtype=jnp.int32)
    rows = starts[:, None] + jnp.arange(W_ROWS, dtype=jnp.int32)[None, :]

    def per_batch(v):
        return v[:, rows].reshape(B * 2, W_ROWS)

    wcat = jnp.concatenate([
        W_atom[:ATOM_ROWS],
        W_in, W_out, W_gt,
        jnp.zeros((1, N_EMBD), dtype=jnp.float32),
    ], axis=0)
    return wcat, per_batch(ia), per_batch(i1), per_batch(i2)


DEAD_REL = VOCAB + 1
DEAD_EDGE = 1 << 14


def _prep_gab_inputs(rel_pos, attn_edge_type, W_rel_pos, W_edge, W_vt):
    aet = attn_edge_type.astype(jnp.int16)

    def shift(v, dead):
        pad = jnp.full((B, 1, N), dead, dtype=jnp.int16)
        s = jnp.concatenate([pad, v[:, : N - 1]], axis=1)
        return s.reshape(B, N_QC, 1, CCOLS)

    def tail(v):
        return v[:, N - 1].reshape(B, 1, N)

    rp = rel_pos.astype(jnp.int16)
    ra, rt = shift(rp, DEAD_REL), tail(rp)
    e = [aet[..., c] for c in range(3)]
    e0, e1, e2 = (shift(v, DEAD_EDGE) for v in e)
    et0, et1, et2 = (tail(v) for v in e)

    wc = jnp.zeros((VOC, N_HEAD), dtype=jnp.float32)
    wc = wc.at[: VOCAB + 1].set(W_rel_pos)
    wc = wc.at[REL_V:].set(W_edge * (1.0 / 3.0))
    wct = wc.T.astype(jnp.bfloat16)
    wvt = W_vt.reshape(1, N_HEAD).T
    return ra, e0, e1, e2, rt, et0, et1, et2, wct, wvt


def kernel(x, y, attn_bias, rel_pos, in_degree, out_degree, edge_input,
           attn_edge_type, W_rel_pos, W_vt, W_edge, W_atom, W_in, W_out,
           W_gt):
    wcat, ia, i1, i2 = _prep_node_inputs(
        x, in_degree, out_degree, W_atom, W_in, W_out, W_gt)
    gnf = _sc_node(wcat, ia, i1, i2)

    ra, e0, e1, e2, rt, et0, et1, et2, wct, wvt = _prep_gab_inputs(
        rel_pos, attn_edge_type, W_rel_pos, W_edge, W_vt)
    gab = _gab_call(ra, e0, e1, e2, rt, et0, et1, et2, attn_bias, wct, wvt)
    return (gnf, gab)


# speedup vs baseline: 18.9660x; 1.2153x over previous
import jax
import jax.numpy as jnp
from jax import lax
from jax.experimental import pallas as pl
from jax.experimental.pallas import tpu as pltpu
from jax.experimental.pallas import tpu_sc as plsc

B = 16
N = 128
FEAT = 9
VOCAB = 128
N_HEAD = 32
N_EMBD = 768
ROWS_PER_B = N * FEAT + FEAT
ATOM_ROWS = N * FEAT

ROW_GT = ATOM_ROWS
CAT_ROWS = ATOM_ROWS + 2
D_ROWS = 2176
D_ZERO = 2048

W_ROWS = 624
CHUNK = 24
N_CHUNKS = W_ROWS // CHUNK
P1_BASE = 536
LAST_ROW = ROWS_PER_B - 1

REL_V = 144
EDGE_V = 512
VOC = REL_V + EDGE_V
QCHUNK = 16
CCOLS = QCHUNK * N
N_QC = N // QCHUNK


def _sc_node_kernel(wcat_hbm, d_hbm, ia_hbm, id_hbm, out_hbm,
                    ia_v, id_v, a0, a1, d0, d1, w0, w1,
                    sa0, sa1, sd0, sd1, sw0, sw1):
    w = lax.axis_index("s") * 2 + lax.axis_index("c")
    b = w // 2
    base = (w % 2) * P1_BASE

    pltpu.sync_copy(ia_hbm.at[w], ia_v)
    pltpu.sync_copy(id_hbm.at[w], id_v)

    abuf = (a0, a1)
    dbuf = (d0, d1)
    wbuf = (w0, w1)
    asem = (sa0, sa1)
    dsem = (sd0, sd1)
    wsem = (sw0, sw1)

    def fire_gather(c):
        s = c % 2
        r0 = c * CHUNK
        ha = pltpu.async_copy(wcat_hbm.at[ia_v.at[pl.ds(r0, CHUNK)]],
                              abuf[s], asem[s])
        hd = pltpu.async_copy(d_hbm.at[id_v.at[pl.ds(r0, CHUNK)]],
                              dbuf[s], dsem[s])
        return ha, hd

    hg = [None, None]
    hw = [None, None]
    hg[0] = fire_gather(0)
    hg[1] = fire_gather(1)
    for c in range(N_CHUNKS):
        s = c % 2
        ha, hd = hg[s]
        ha.wait()
        hd.wait()
        if hw[s] is not None:
            hw[s].wait()

        @pl.loop(0, CHUNK)
        def _row(r, s=s):
            @pl.loop(0, N_EMBD, step=64)
            def _vec(j, r=r, s=s):
                for u in range(4):
                    sl = pl.ds(j + u * 16, 16)
                    wbuf[s][r, sl] = abuf[s][r, sl] + dbuf[s][r, sl]

        if c + 2 < N_CHUNKS:
            hg[s] = fire_gather(c + 2)
        hw[s] = pltpu.async_copy(wbuf[s],
                                 out_hbm.at[b, pl.ds(base + c * CHUNK, CHUNK)],
                                 wsem[s])
    hw[0].wait()
    hw[1].wait()

    @pl.when(w % 2 == 1)
    def _last_row():
        pltpu.sync_copy(wcat_hbm.at[pl.ds(ROW_GT, 1)],
                        out_hbm.at[b, pl.ds(LAST_ROW, 1)])


def _sc_node(wcat, d, ia, idd):
    mesh = plsc.VectorSubcoreMesh(core_axis_name="c", subcore_axis_name="s")
    buf = pltpu.VMEM((CHUNK, N_EMBD), jnp.float32)
    kern = pl.kernel(
        _sc_node_kernel,
        out_type=jax.ShapeDtypeStruct((B, ROWS_PER_B, N_EMBD), jnp.float32),
        mesh=mesh,
        scratch_types=[
            pltpu.VMEM((W_ROWS,), jnp.int32),
            pltpu.VMEM((W_ROWS,), jnp.int32),
            buf, buf, buf, buf, buf, buf,
        ] + [pltpu.SemaphoreType.DMA] * 6,
    )
    return kern(wcat, d, ia, idd)


def _multi_hot(ra, e0, e1, e2, cols):
    one = jnp.bfloat16(1.0)
    zero = jnp.bfloat16(0.0)
    iv_r = lax.broadcasted_iota(jnp.int16, (REL_V, cols), 0)
    oh_rel = jnp.where(iv_r == ra, one, zero)
    iv_e = lax.broadcasted_iota(jnp.int16, (EDGE_V, cols), 0)
    oh_e = (jnp.where(iv_e == e0, one, zero)
            + jnp.where(iv_e == e1, one, zero)
            + jnp.where(iv_e == e2, one, zero))
    return jnp.concatenate([oh_rel, oh_e], axis=0)


def _gab_body(ra_ref, e0_ref, e1_ref, e2_ref, rt_ref, et0_ref, et1_ref,
              et2_ref, ab_ref, wc_ref, wvt_ref, out_ref):
    kc = pl.program_id(1)

    oh = _multi_hot(ra_ref[0, 0], e0_ref[0, 0], e1_ref[0, 0], e2_ref[0, 0],
                    CCOLS)
    acc = lax.dot_general(wc_ref[...], oh, (((1,), (0,)), ((), ())),
                          preferred_element_type=jnp.float32)
    t = acc.reshape(N_HEAD, QCHUNK, N)

    abq = ab_ref[0, pl.ds(kc * QCHUNK, QCHUNK), :]
    notrow0 = (lax.broadcasted_iota(jnp.int32, (QCHUNK, 1), 0)
               + kc * QCHUNK) > 0
    vtc = jnp.where(notrow0[None, :, :], wvt_ref[...][:, None, :], 0.0)
    col0 = 2.0 * abq[None, :, 0:1] + vtc
    inter = 2.0 * abq[None, :, 1:] + t
    rows = jnp.concatenate([col0, inter], axis=2)
    out_ref[0, :, pl.ds(kc * QCHUNK, QCHUNK), :] = rows

    @pl.when(kc == N_QC - 1)
    def _tail():
        oh_t = _multi_hot(rt_ref[0, 0], et0_ref[0, 0], et1_ref[0, 0],
                          et2_ref[0, 0], N)
        acc_t = lax.dot_general(wc_ref[...], oh_t, (((1,), (0,)), ((), ())),
                                preferred_element_type=jnp.float32)
        ab_l = ab_ref[0, pl.ds(N, 1), :]
        c0 = 2.0 * ab_l[:, 0:1] + wvt_ref[...]
        it = 2.0 * ab_l[:, 1:] + acc_t
        out_ref[0, :, N, :] = jnp.concatenate([c0, it], axis=1)


def _gab_call(ra, e0, e1, e2, rt, et0, et1, et2, ab, wct, wvt,
              interpret=False):
    idx_spec = pl.BlockSpec((1, 1, 1, CCOLS), lambda b, k: (b, k, 0, 0))
    tail_spec = pl.BlockSpec((1, 1, N), lambda b, k: (b, 0, 0))
    return pl.pallas_call(
        _gab_body,
        grid=(B, N_QC),
        in_specs=[
            idx_spec, idx_spec, idx_spec, idx_spec,
            tail_spec, tail_spec, tail_spec, tail_spec,
            pl.BlockSpec((1, N + 1, N + 1), lambda b, k: (b, 0, 0)),
            pl.BlockSpec((N_HEAD, VOC), lambda b, k: (0, 0)),
            pl.BlockSpec((N_HEAD, 1), lambda b, k: (0, 0)),
        ],
        out_specs=pl.BlockSpec((1, N_HEAD, N + 1, N + 1),
                               lambda b, k: (b, 0, 0, 0)),
        out_shape=jax.ShapeDtypeStruct((B, N_HEAD, N + 1, N + 1), jnp.float32),
        interpret=interpret,
    )(ra, e0, e1, e2, rt, et0, et1, et2, ab, wct, wvt)


def _d_body(ind_ref, outd_ref, wio_ref, out_ref):
    bidx = pl.program_id(0)
    iv = lax.broadcasted_iota(jnp.int16, (N, 1024), 1)
    one = jnp.bfloat16(1.0)
    zero = jnp.bfloat16(0.0)
    oh = (jnp.where(iv == ind_ref[0], one, zero)
          + jnp.where(iv == outd_ref[0], one, zero))
    d = lax.dot_general(oh, wio_ref[...], (((1,), (0,)), ((), ())),
                        preferred_element_type=jnp.float32)
    out_ref[...] = d * jnp.where(bidx < B, 1.0, 0.0)


def _d_call(ind_p, outd_p, wio):
    return pl.pallas_call(
        _d_body,
        grid=(B + 1,),
        in_specs=[
            pl.BlockSpec((1, N, 1), lambda b: (b, 0, 0)),
            pl.BlockSpec((1, N, 1), lambda b: (b, 0, 0)),
            pl.BlockSpec((1024, N_EMBD), lambda b: (0, 0)),
        ],
        out_specs=pl.BlockSpec((N, N_EMBD), lambda b: (b, 0)),
        out_shape=jax.ShapeDtypeStruct((D_ROWS, N_EMBD), jnp.float32),
    )(ind_p, outd_p, wio)


def _prep_node_inputs(x, in_degree, out_degree, W_atom, W_in, W_out, W_gt):
    x = x.astype(jnp.int32)
    t = jnp.arange(ROWS_PER_B, dtype=jnp.int32)
    node = jnp.minimum(t // FEAT, N - 1)
    feat = t % FEAT
    is_atom = t < ATOM_ROWS

    xo = x + (jnp.arange(FEAT, dtype=jnp.int32) * VOCAB)
    a_rows = xo[:, node, feat]
    ia = jnp.where(is_atom[None, :], a_rows, ROW_GT)
    bnode = (jnp.arange(B, dtype=jnp.int32) * N)[:, None] + node[None, :]
    idd = jnp.where(is_atom[None, :], bnode, D_ZERO)

    starts = jnp.array([0, P1_BASE], dtype=jnp.int32)
    rows = starts[:, None] + jnp.arange(W_ROWS, dtype=jnp.int32)[None, :]

    def per_batch(v):
        return v[:, rows].reshape(B * 2, W_ROWS)

    wcat = jnp.concatenate([
        W_atom[:ATOM_ROWS],
        W_gt,
        jnp.zeros((1, N_EMBD), dtype=jnp.float32),
    ], axis=0)

    pad = jnp.zeros((1, N), dtype=jnp.int16)
    ind_p = jnp.concatenate(
        [in_degree.astype(jnp.int16), pad], axis=0).reshape(B + 1, N, 1)
    outd_p = jnp.concatenate(
        [out_degree.astype(jnp.int16) + 512, pad], axis=0).reshape(B + 1, N, 1)
    wio = jnp.concatenate([W_in, W_out], axis=0).astype(jnp.bfloat16)
    return wcat, per_batch(ia), per_batch(idd), ind_p, outd_p, wio


DEAD_REL = VOCAB + 1
DEAD_EDGE = 1 << 14


def _prep_gab_inputs(rel_pos, attn_edge_type, W_rel_pos, W_edge, W_vt):
    aet = attn_edge_type.astype(jnp.int16)

    def shift(v, dead):
        pad = jnp.full((B, 1, N), dead, dtype=jnp.int16)
        s = jnp.concatenate([pad, v[:, : N - 1]], axis=1)
        return s.reshape(B, N_QC, 1, CCOLS)

    def tail(v):
        return v[:, N - 1].reshape(B, 1, N)

    rp = rel_pos.astype(jnp.int16)
    ra, rt = shift(rp, DEAD_REL), tail(rp)
    e = [aet[..., c] for c in range(3)]
    e0, e1, e2 = (shift(v, DEAD_EDGE) for v in e)
    et0, et1, et2 = (tail(v) for v in e)

    wc = jnp.zeros((VOC, N_HEAD), dtype=jnp.float32)
    wc = wc.at[: VOCAB + 1].set(W_rel_pos)
    wc = wc.at[REL_V:].set(W_edge * (1.0 / 3.0))
    wct = wc.T.astype(jnp.bfloat16)
    wvt = W_vt.reshape(1, N_HEAD).T
    return ra, e0, e1, e2, rt, et0, et1, et2, wct, wvt


def kernel(x, y, attn_bias, rel_pos, in_degree, out_degree, edge_input,
           attn_edge_type, W_rel_pos, W_vt, W_edge, W_atom, W_in, W_out,
           W_gt):
    wcat, ia, idd, ind_p, outd_p, wio = _prep_node_inputs(
        x, in_degree, out_degree, W_atom, W_in, W_out, W_gt)
    d = _d_call(ind_p, outd_p, wio)
    gnf = _sc_node(wcat, d, ia, idd)

    ra, e0, e1, e2, rt, et0, et1, et2, wct, wvt = _prep_gab_inputs(
        rel_pos, attn_edge_type, W_rel_pos, W_edge, W_vt)
    gab = _gab_call(ra, e0, e1, e2, rt, et0, et1, et2, attn_bias, wct, wvt)
    return (gnf, gab)


# speedup vs baseline: 28.3573x; 1.4952x over previous
import jax
import jax.numpy as jnp
from jax import lax
from jax.experimental import pallas as pl
from jax.experimental.pallas import tpu as pltpu
from jax.experimental.pallas import tpu_sc as plsc

B = 16
N = 128
FEAT = 9
VOCAB = 128
N_HEAD = 32
N_EMBD = 768
ROWS_PER_B = N * FEAT + FEAT
ATOM_ROWS = N * FEAT

ROW_GT = ATOM_ROWS
CAT_ROWS = ATOM_ROWS + 2
D_ROWS = 2176
D_ZERO = 2048

W_ROWS = 624
CHUNK = 24
N_CHUNKS = W_ROWS // CHUNK
P1_BASE = 536
LAST_ROW = ROWS_PER_B - 1

REL_V = 144
EDGE_V = 512
VOC = REL_V + EDGE_V
QCHUNK = 16
CCOLS = QCHUNK * N
N_QC = N // QCHUNK


def _sc_node_kernel(wcat_hbm, d_hbm, ia_hbm, id_hbm, out_hbm,
                    ia_v, id_v, a0, a1, d0, d1, w0, w1,
                    sa0, sa1, sd0, sd1, sw0, sw1):
    w = lax.axis_index("s") * 2 + lax.axis_index("c")
    b = w // 2
    base = (w % 2) * P1_BASE

    pltpu.sync_copy(ia_hbm.at[w], ia_v)
    pltpu.sync_copy(id_hbm.at[w], id_v)

    abuf = (a0, a1)
    dbuf = (d0, d1)
    wbuf = (w0, w1)
    asem = (sa0, sa1)
    dsem = (sd0, sd1)
    wsem = (sw0, sw1)

    def fire_gather(c):
        s = c % 2
        r0 = c * CHUNK
        ha = pltpu.async_copy(wcat_hbm.at[ia_v.at[pl.ds(r0, CHUNK)]],
                              abuf[s], asem[s])
        hd = pltpu.async_copy(d_hbm.at[id_v.at[pl.ds(r0, CHUNK)]],
                              dbuf[s], dsem[s])
        return ha, hd

    hg = [None, None]
    hw = [None, None]
    hg[0] = fire_gather(0)
    hg[1] = fire_gather(1)
    for c in range(N_CHUNKS):
        s = c % 2
        ha, hd = hg[s]
        ha.wait()
        hd.wait()
        if hw[s] is not None:
            hw[s].wait()

        @pl.loop(0, CHUNK)
        def _row(r, s=s):
            @pl.loop(0, N_EMBD, step=192)
            def _vec(j, r=r, s=s):
                for u in range(12):
                    sl = pl.ds(j + u * 16, 16)
                    wbuf[s][r, sl] = abuf[s][r, sl] + dbuf[s][r, sl]

        if c + 2 < N_CHUNKS:
            hg[s] = fire_gather(c + 2)
        hw[s] = pltpu.async_copy(wbuf[s],
                                 out_hbm.at[pl.ds(base + c * CHUNK, CHUNK), b],
                                 wsem[s])
    hw[0].wait()
    hw[1].wait()

    @pl.when(w % 2 == 1)
    def _last_row():
        pltpu.sync_copy(wcat_hbm.at[pl.ds(ROW_GT, 1)],
                        out_hbm.at[pl.ds(LAST_ROW, 1), b])


def _sc_node(wcat, d, ia, idd):
    mesh = plsc.VectorSubcoreMesh(core_axis_name="c", subcore_axis_name="s")
    buf = pltpu.VMEM((CHUNK, N_EMBD), jnp.float32)
    kern = pl.kernel(
        _sc_node_kernel,
        out_type=jax.ShapeDtypeStruct((ROWS_PER_B, B, N_EMBD), jnp.float32),
        mesh=mesh,
        scratch_types=[
            pltpu.VMEM((W_ROWS,), jnp.int32),
            pltpu.VMEM((W_ROWS,), jnp.int32),
            buf, buf, buf, buf, buf, buf,
        ] + [pltpu.SemaphoreType.DMA] * 6,
    )
    return kern(wcat, d, ia, idd)


def _multi_hot(ra, e0, e1, e2, cols):
    one = jnp.bfloat16(1.0)
    zero = jnp.bfloat16(0.0)
    iv_r = lax.broadcasted_iota(jnp.int16, (REL_V, cols), 0)
    oh_rel = jnp.where(iv_r == ra, one, zero)
    iv_e = lax.broadcasted_iota(jnp.int16, (EDGE_V, cols), 0)
    oh_e = (jnp.where(iv_e == e0, one, zero)
            + jnp.where(iv_e == e1, one, zero)
            + jnp.where(iv_e == e2, one, zero))
    return jnp.concatenate([oh_rel, oh_e], axis=0)


def _gab_body(ra_ref, e0_ref, e1_ref, e2_ref, rt_ref, et0_ref, et1_ref,
              et2_ref, ab_ref, wc_ref, wvt_ref, out_ref):
    kc = pl.program_id(1)

    oh = _multi_hot(ra_ref[0, 0], e0_ref[0, 0], e1_ref[0, 0], e2_ref[0, 0],
                    CCOLS)
    acc = lax.dot_general(wc_ref[...], oh, (((1,), (0,)), ((), ())),
                          preferred_element_type=jnp.float32)
    t = jnp.swapaxes(acc.reshape(N_HEAD, QCHUNK, N), 0, 1)

    abq = ab_ref[0, pl.ds(kc * QCHUNK, QCHUNK), :]
    notrow0 = (lax.broadcasted_iota(jnp.int32, (QCHUNK, 1, 1), 0)
               + kc * QCHUNK) > 0
    vtc = jnp.where(notrow0, wvt_ref[...].reshape(1, N_HEAD, 1), 0.0)
    col0 = 2.0 * abq[:, None, 0:1] + vtc
    inter = 2.0 * abq[:, None, 1:] + t
    rows = jnp.concatenate([col0, inter], axis=2)
    out_ref[0, pl.ds(kc * QCHUNK, QCHUNK), :, :] = rows

    @pl.when(kc == N_QC - 1)
    def _tail():
        oh_t = _multi_hot(rt_ref[0, 0], et0_ref[0, 0], et1_ref[0, 0],
                          et2_ref[0, 0], N)
        acc_t = lax.dot_general(wc_ref[...], oh_t, (((1,), (0,)), ((), ())),
                                preferred_element_type=jnp.float32)
        ab_l = ab_ref[0, pl.ds(N, 1), :]
        c0 = 2.0 * ab_l[:, 0:1] + wvt_ref[...]
        it = 2.0 * ab_l[:, 1:] + acc_t
        out_ref[0, N, :, :] = jnp.concatenate([c0, it], axis=1)


def _gab_call(ra, e0, e1, e2, rt, et0, et1, et2, ab, wct, wvt,
              interpret=False):
    idx_spec = pl.BlockSpec((1, 1, 1, CCOLS), lambda b, k: (b, k, 0, 0))
    tail_spec = pl.BlockSpec((1, 1, N), lambda b, k: (b, 0, 0))
    return pl.pallas_call(
        _gab_body,
        grid=(B, N_QC),
        in_specs=[
            idx_spec, idx_spec, idx_spec, idx_spec,
            tail_spec, tail_spec, tail_spec, tail_spec,
            pl.BlockSpec((1, N + 1, N + 1), lambda b, k: (b, 0, 0)),
            pl.BlockSpec((N_HEAD, VOC), lambda b, k: (0, 0)),
            pl.BlockSpec((N_HEAD, 1), lambda b, k: (0, 0)),
        ],
        out_specs=pl.BlockSpec((1, N + 1, N_HEAD, N + 1),
                               lambda b, k: (b, 0, 0, 0)),
        out_shape=jax.ShapeDtypeStruct((B, N + 1, N_HEAD, N + 1), jnp.float32),
        interpret=interpret,
    )(ra, e0, e1, e2, rt, et0, et1, et2, ab, wct, wvt)


def _d_body(ind_ref, outd_ref, wio_ref, out_ref):
    bidx = pl.program_id(0)
    iv = lax.broadcasted_iota(jnp.int16, (N, 1024), 1)
    one = jnp.bfloat16(1.0)
    zero = jnp.bfloat16(0.0)
    oh = (jnp.where(iv == ind_ref[0], one, zero)
          + jnp.where(iv == outd_ref[0], one, zero))
    d = lax.dot_general(oh, wio_ref[...], (((1,), (0,)), ((), ())),
                        preferred_element_type=jnp.float32)
    out_ref[...] = d * jnp.where(bidx < B, 1.0, 0.0)


def _d_call(ind_p, outd_p, wio):
    return pl.pallas_call(
        _d_body,
        grid=(B + 1,),
        in_specs=[
            pl.BlockSpec((1, N, 1), lambda b: (b, 0, 0)),
            pl.BlockSpec((1, N, 1), lambda b: (b, 0, 0)),
            pl.BlockSpec((1024, N_EMBD), lambda b: (0, 0)),
        ],
        out_specs=pl.BlockSpec((N, N_EMBD), lambda b: (b, 0)),
        out_shape=jax.ShapeDtypeStruct((D_ROWS, N_EMBD), jnp.float32),
    )(ind_p, outd_p, wio)


def _prep_node_inputs(x, in_degree, out_degree, W_atom, W_in, W_out, W_gt):
    x = x.astype(jnp.int32)
    xo = x + (jnp.arange(FEAT, dtype=jnp.int32) * VOCAB)
    gt_ia = jnp.full((B, FEAT), ROW_GT, dtype=jnp.int32)
    ia = jnp.concatenate([xo.reshape(B, ATOM_ROWS), gt_ia], axis=1)

    bnode = ((jnp.arange(B, dtype=jnp.int32) * N)[:, None, None]
             + jnp.arange(N, dtype=jnp.int32)[None, :, None])
    bnode = jnp.broadcast_to(bnode, (B, N, FEAT)).reshape(B, ATOM_ROWS)
    gt_id = jnp.full((B, FEAT), D_ZERO, dtype=jnp.int32)
    idd = jnp.concatenate([bnode, gt_id], axis=1)

    def per_batch(v):
        return jnp.stack(
            [v[:, :W_ROWS], v[:, P1_BASE:P1_BASE + W_ROWS]],
            axis=1).reshape(B * 2, W_ROWS)

    wcat = jnp.concatenate([
        W_atom[:ATOM_ROWS],
        W_gt,
        jnp.zeros((1, N_EMBD), dtype=jnp.float32),
    ], axis=0)

    pad = jnp.zeros((1, N), dtype=jnp.int16)
    ind_p = jnp.concatenate(
        [in_degree.astype(jnp.int16), pad], axis=0).reshape(B + 1, N, 1)
    outd_p = jnp.concatenate(
        [out_degree.astype(jnp.int16) + 512, pad], axis=0).reshape(B + 1, N, 1)
    wio = jnp.concatenate([W_in, W_out], axis=0).astype(jnp.bfloat16)
    return wcat, per_batch(ia), per_batch(idd), ind_p, outd_p, wio


DEAD_REL = VOCAB + 1
DEAD_EDGE = 1 << 14


def _prep_gab_inputs(rel_pos, attn_edge_type, W_rel_pos, W_edge, W_vt):
    aet = attn_edge_type.astype(jnp.int16)

    def shift(v, dead):
        pad = jnp.full((B, 1, N), dead, dtype=jnp.int16)
        s = jnp.concatenate([pad, v[:, : N - 1]], axis=1)
        return s.reshape(B, N_QC, 1, CCOLS)

    def tail(v):
        return v[:, N - 1].reshape(B, 1, N)

    rp = rel_pos.astype(jnp.int16)
    ra, rt = shift(rp, DEAD_REL), tail(rp)
    e = [aet[..., c] for c in range(3)]
    e0, e1, e2 = (shift(v, DEAD_EDGE) for v in e)
    et0, et1, et2 = (tail(v) for v in e)

    wc = jnp.zeros((VOC, N_HEAD), dtype=jnp.float32)
    wc = wc.at[: VOCAB + 1].set(W_rel_pos)
    wc = wc.at[REL_V:].set(W_edge * (1.0 / 3.0))
    wct = wc.T.astype(jnp.bfloat16)
    wvt = W_vt.reshape(1, N_HEAD).T
    return ra, e0, e1, e2, rt, et0, et1, et2, wct, wvt


def kernel(x, y, attn_bias, rel_pos, in_degree, out_degree, edge_input,
           attn_edge_type, W_rel_pos, W_vt, W_edge, W_atom, W_in, W_out,
           W_gt):
    wcat, ia, idd, ind_p, outd_p, wio = _prep_node_inputs(
        x, in_degree, out_degree, W_atom, W_in, W_out, W_gt)
    d = _d_call(ind_p, outd_p, wio)
    gnf_t = _sc_node(wcat, d, ia, idd)
    gnf = jnp.transpose(gnf_t, (1, 0, 2))

    ra, e0, e1, e2, rt, et0, et1, et2, wct, wvt = _prep_gab_inputs(
        rel_pos, attn_edge_type, W_rel_pos, W_edge, W_vt)
    gab_t = _gab_call(ra, e0, e1, e2, rt, et0, et1, et2, attn_bias, wct, wvt)
    gab = jnp.transpose(gab_t, (0, 2, 1, 3))
    return (gnf, gab)


# speedup vs baseline: 32.1590x; 1.1341x over previous
import jax
import jax.numpy as jnp
from jax import lax
from jax.experimental import pallas as pl
from jax.experimental.pallas import tpu as pltpu
from jax.experimental.pallas import tpu_sc as plsc

B = 16
N = 128
FEAT = 9
VOCAB = 128
N_HEAD = 32
N_EMBD = 768
ROWS_PER_B = N * FEAT + FEAT
ATOM_ROWS = N * FEAT

ROW_GT = ATOM_ROWS
CAT_ROWS = ATOM_ROWS + 2
D_ROWS = 2176
D_ZERO = 2048

W_ROWS = 624
CHUNK = 24
N_CHUNKS = W_ROWS // CHUNK
P1_BASE = 536
LAST_ROW = ROWS_PER_B - 1

REL_V = 144
EDGE_V = 512
VOC = REL_V + EDGE_V
QCHUNK = 32
CCOLS = QCHUNK * N
N_QC = N // QCHUNK


def _sc_node_kernel(wcat_hbm, d_hbm, wgt_hbm, ia_hbm, id_hbm, out_hbm,
                    ia_v, id_v, a0, a1, d0, d1, w0, w1,
                    sa0, sa1, sd0, sd1, sw0, sw1):
    w = lax.axis_index("s") * 2 + lax.axis_index("c")
    b = w // 2
    base = (w % 2) * P1_BASE

    pltpu.sync_copy(ia_hbm.at[w], ia_v)
    pltpu.sync_copy(id_hbm.at[w], id_v)

    abuf = (a0, a1)
    dbuf = (d0, d1)
    wbuf = (w0, w1)
    asem = (sa0, sa1)
    dsem = (sd0, sd1)
    wsem = (sw0, sw1)

    def fire_gather(c):
        s = c % 2
        r0 = c * CHUNK
        ha = pltpu.async_copy(wcat_hbm.at[ia_v.at[pl.ds(r0, CHUNK)]],
                              abuf[s], asem[s])
        hd = pltpu.async_copy(d_hbm.at[id_v.at[pl.ds(r0, CHUNK)]],
                              dbuf[s], dsem[s])
        return ha, hd

    hg = [None, None]
    hw = [None, None]
    hg[0] = fire_gather(0)
    hg[1] = fire_gather(1)
    for c in range(N_CHUNKS):
        s = c % 2
        ha, hd = hg[s]
        ha.wait()
        hd.wait()
        if hw[s] is not None:
            hw[s].wait()

        @pl.loop(0, CHUNK)
        def _row(r, s=s):
            @pl.loop(0, N_EMBD, step=192)
            def _vec(j, r=r, s=s):
                for u in range(12):
                    sl = pl.ds(j + u * 16, 16)
                    wbuf[s][r, sl] = abuf[s][r, sl] + dbuf[s][r, sl]

        if c + 2 < N_CHUNKS:
            hg[s] = fire_gather(c + 2)
        hw[s] = pltpu.async_copy(wbuf[s],
                                 out_hbm.at[pl.ds(base + c * CHUNK, CHUNK), b],
                                 wsem[s])
    hw[0].wait()
    hw[1].wait()

    @pl.when(w % 2 == 1)
    def _last_row():
        pltpu.sync_copy(wgt_hbm, out_hbm.at[pl.ds(LAST_ROW, 1), b])


def _sc_node(wcat, d, wgt, ia, idd):
    mesh = plsc.VectorSubcoreMesh(core_axis_name="c", subcore_axis_name="s")
    buf = pltpu.VMEM((CHUNK, N_EMBD), jnp.float32)
    kern = pl.kernel(
        _sc_node_kernel,
        out_type=jax.ShapeDtypeStruct((ROWS_PER_B, B, N_EMBD), jnp.float32),
        mesh=mesh,
        scratch_types=[
            pltpu.VMEM((W_ROWS,), jnp.int32),
            pltpu.VMEM((W_ROWS,), jnp.int32),
            buf, buf, buf, buf, buf, buf,
        ] + [pltpu.SemaphoreType.DMA] * 6,
    )
    return kern(wcat, d, wgt, ia, idd)


def _multi_hot(ra, e0, e1, e2, cols):
    one = jnp.bfloat16(1.0)
    zero = jnp.bfloat16(0.0)
    iv_r = lax.broadcasted_iota(jnp.int16, (REL_V, cols), 0)
    oh_rel = jnp.where(iv_r == ra, one, zero)
    iv_e = lax.broadcasted_iota(jnp.int16, (EDGE_V, cols), 0)
    oh_e = (jnp.where(iv_e == e0, one, zero)
            + jnp.where(iv_e == e1, one, zero)
            + jnp.where(iv_e == e2, one, zero))
    return jnp.concatenate([oh_rel, oh_e], axis=0)


def _gab_body(ra_ref, e0_ref, e1_ref, e2_ref, rt_ref, et0_ref, et1_ref,
              et2_ref, ab_ref, wc_ref, wvt_ref, out_ref):
    kc = pl.program_id(1)

    oh = _multi_hot(ra_ref[0, 0], e0_ref[0, 0], e1_ref[0, 0], e2_ref[0, 0],
                    CCOLS)
    acc = lax.dot_general(wc_ref[...], oh, (((1,), (0,)), ((), ())),
                          preferred_element_type=jnp.float32)
    t = jnp.swapaxes(acc.reshape(N_HEAD, QCHUNK, N), 0, 1)

    abq = ab_ref[0, pl.ds(kc * QCHUNK, QCHUNK), :]
    notrow0 = (lax.broadcasted_iota(jnp.int32, (QCHUNK, 1, 1), 0)
               + kc * QCHUNK) > 0
    vtc = jnp.where(notrow0, wvt_ref[...].reshape(1, N_HEAD, 1), 0.0)
    col0 = 2.0 * abq[:, None, 0:1] + vtc
    inter = 2.0 * abq[:, None, 1:] + t
    rows = jnp.concatenate([col0, inter], axis=2)
    out_ref[0, pl.ds(kc * QCHUNK, QCHUNK), :, :] = rows

    @pl.when(kc == N_QC - 1)
    def _tail():
        oh_t = _multi_hot(rt_ref[0, 0], et0_ref[0, 0], et1_ref[0, 0],
                          et2_ref[0, 0], N)
        acc_t = lax.dot_general(wc_ref[...], oh_t, (((1,), (0,)), ((), ())),
                                preferred_element_type=jnp.float32)
        ab_l = ab_ref[0, pl.ds(N, 1), :]
        c0 = 2.0 * ab_l[:, 0:1] + wvt_ref[...]
        it = 2.0 * ab_l[:, 1:] + acc_t
        out_ref[0, N, :, :] = jnp.concatenate([c0, it], axis=1)


def _gab_call(ra, e0, e1, e2, rt, et0, et1, et2, ab, wct, wvt,
              interpret=False):
    idx_spec = pl.BlockSpec((1, 1, 1, CCOLS), lambda b, k: (b, k, 0, 0))
    tail_spec = pl.BlockSpec((1, 1, N), lambda b, k: (b, 0, 0))
    return pl.pallas_call(
        _gab_body,
        grid=(B, N_QC),
        in_specs=[
            idx_spec, idx_spec, idx_spec, idx_spec,
            tail_spec, tail_spec, tail_spec, tail_spec,
            pl.BlockSpec((1, N + 1, N + 1), lambda b, k: (b, 0, 0)),
            pl.BlockSpec((N_HEAD, VOC), lambda b, k: (0, 0)),
            pl.BlockSpec((N_HEAD, 1), lambda b, k: (0, 0)),
        ],
        out_specs=pl.BlockSpec((1, N + 1, N_HEAD, N + 1),
                               lambda b, k: (b, 0, 0, 0)),
        out_shape=jax.ShapeDtypeStruct((B, N + 1, N_HEAD, N + 1), jnp.float32),
        interpret=interpret,
    )(ra, e0, e1, e2, rt, et0, et1, et2, ab, wct, wvt)


def _d_body(ind_ref, outd_ref, wio_ref, gt0_ref, out_ref):
    bidx = pl.program_id(0)

    @pl.when(bidx < B)
    def _():
        iv = lax.broadcasted_iota(jnp.int16, (N, 1024), 1)
        one = jnp.bfloat16(1.0)
        zero = jnp.bfloat16(0.0)
        oh = (jnp.where(iv == ind_ref[0], one, zero)
              + jnp.where(iv == outd_ref[0], one, zero))
        out_ref[...] = lax.dot_general(oh, wio_ref[...],
                                       (((1,), (0,)), ((), ())),
                                       preferred_element_type=jnp.float32)

    @pl.when(bidx == B)
    def _():
        out_ref[...] = jnp.broadcast_to(gt0_ref[...], (N, N_EMBD))


def _d_call(ind_p, outd_p, wio, gt0):
    return pl.pallas_call(
        _d_body,
        grid=(B + 1,),
        in_specs=[
            pl.BlockSpec((1, N, 1), lambda b: (b, 0, 0)),
            pl.BlockSpec((1, N, 1), lambda b: (b, 0, 0)),
            pl.BlockSpec((1024, N_EMBD), lambda b: (0, 0)),
            pl.BlockSpec((1, N_EMBD), lambda b: (0, 0)),
        ],
        out_specs=pl.BlockSpec((N, N_EMBD), lambda b: (b, 0)),
        out_shape=jax.ShapeDtypeStruct((D_ROWS, N_EMBD), jnp.float32),
    )(ind_p, outd_p, wio, gt0)


def _prep_node_inputs(x, in_degree, out_degree, W_atom, W_in, W_out, W_gt):
    x = x.astype(jnp.int32)
    xo = x + (jnp.arange(FEAT, dtype=jnp.int32) * VOCAB)
    gt_ia = jnp.zeros((B, FEAT), dtype=jnp.int32)
    ia = jnp.concatenate([xo.reshape(B, ATOM_ROWS), gt_ia], axis=1)

    bnode = ((jnp.arange(B, dtype=jnp.int32) * N)[:, None, None]
             + jnp.arange(N, dtype=jnp.int32)[None, :, None])
    bnode = jnp.broadcast_to(bnode, (B, N, FEAT)).reshape(B, ATOM_ROWS)
    gt_id = jnp.full((B, FEAT), D_ZERO, dtype=jnp.int32)
    idd = jnp.concatenate([bnode, gt_id], axis=1)

    def per_batch(v):
        return jnp.stack(
            [v[:, :W_ROWS], v[:, P1_BASE:P1_BASE + W_ROWS]],
            axis=1).reshape(B * 2, W_ROWS)

    pad = jnp.zeros((1, N), dtype=jnp.int16)
    ind_p = jnp.concatenate(
        [in_degree.astype(jnp.int16), pad], axis=0).reshape(B + 1, N, 1)
    outd_p = jnp.concatenate(
        [out_degree.astype(jnp.int16) + 512, pad], axis=0).reshape(B + 1, N, 1)
    wio = jnp.concatenate([W_in, W_out], axis=0).astype(jnp.bfloat16)
    gt0 = W_gt - W_atom[0:1]
    return per_batch(ia), per_batch(idd), ind_p, outd_p, wio, gt0


DEAD_REL = VOCAB + 1
DEAD_EDGE = 1 << 14


def _prep_gab_inputs(rel_pos, attn_edge_type, W_rel_pos, W_edge, W_vt):
    aet = attn_edge_type.astype(jnp.int16)

    def shift(v, dead):
        pad = jnp.full((B, 1, N), dead, dtype=jnp.int16)
        s = jnp.concatenate([pad, v[:, : N - 1]], axis=1)
        return s.reshape(B, N_QC, 1, CCOLS)

    def tail(v):
        return v[:, N - 1].reshape(B, 1, N)

    rp = rel_pos.astype(jnp.int16)
    ra, rt = shift(rp, DEAD_REL), tail(rp)
    e = [aet[..., c] for c in range(3)]
    e0, e1, e2 = (shift(v, DEAD_EDGE) for v in e)
    et0, et1, et2 = (tail(v) for v in e)

    wc = jnp.zeros((VOC, N_HEAD), dtype=jnp.float32)
    wc = wc.at[: VOCAB + 1].set(W_rel_pos)
    wc = wc.at[REL_V:].set(W_edge * (1.0 / 3.0))
    wct = wc.T.astype(jnp.bfloat16)
    wvt = W_vt.reshape(1, N_HEAD).T
    return ra, e0, e1, e2, rt, et0, et1, et2, wct, wvt


def kernel(x, y, attn_bias, rel_pos, in_degree, out_degree, edge_input,
           attn_edge_type, W_rel_pos, W_vt, W_edge, W_atom, W_in, W_out,
           W_gt):
    ia, idd, ind_p, outd_p, wio, gt0 = _prep_node_inputs(
        x, in_degree, out_degree, W_atom, W_in, W_out, W_gt)
    d = _d_call(ind_p, outd_p, wio, gt0)
    gnf_t = _sc_node(W_atom, d, W_gt, ia, idd)
    gnf = jnp.transpose(gnf_t, (1, 0, 2))

    ra, e0, e1, e2, rt, et0, et1, et2, wct, wvt = _prep_gab_inputs(
        rel_pos, attn_edge_type, W_rel_pos, W_edge, W_vt)
    gab_t = _gab_call(ra, e0, e1, e2, rt, et0, et1, et2, attn_bias, wct, wvt)
    gab = jnp.transpose(gab_t, (0, 2, 1, 3))
    return (gnf, gab)
